# Initial kernel scaffold; baseline (speedup 1.0000x reference)
import jax
import jax.numpy as jnp
from jax.experimental import pallas as pl


def kernel(x, support_examples, support_labels, num_shots, class_prototypes, prototype_counts):
    raise NotImplementedError("write your pallas kernel here")


# speedup vs baseline: 1.4458x; 1.4458x over previous
import jax
import jax.numpy as jnp
from jax.experimental import pallas as pl
from jax.experimental.pallas import tpu as pltpu

_C_PAD = 1024
_S_BLK = 512
_F_BLK = 2048


def _segsum_body(labels_ref, flat_ref, sums_ref, counts_ref):
    f = pl.program_id(0)
    s = pl.program_id(1)
    lbl = jnp.clip(labels_ref[0, 0, :], 0, _C_PAD - 1)
    cls = jax.lax.broadcasted_iota(jnp.int32, (_C_PAD, _S_BLK), 0)
    onehot = (lbl[None, :] == cls).astype(jnp.float32)
    partial = jnp.dot(onehot, flat_ref[...],
                      preferred_element_type=jnp.float32)

    @pl.when(s == 0)
    def _():
        sums_ref[...] = partial

    @pl.when(s != 0)
    def _():
        sums_ref[...] += partial

    @pl.when(f == 0)
    def _():
        cnt = jnp.sum(onehot, axis=1, keepdims=True)
        cb = jnp.broadcast_to(cnt, (_C_PAD, 128))

        @pl.when(s == 0)
        def _():
            counts_ref[...] = cb

        @pl.when(s != 0)
        def _():
            counts_ref[...] += cb


def _ema_body(sums_ref, counts_ref, protos_ref, pc_ref, out_ref):
    cnt = counts_ref[:, 0:1]
    a = 1.0 / (pc_ref[:, 0:1] + 1.0)
    mean = sums_ref[...] / jnp.maximum(cnt, 1.0)
    upd = (1.0 - a) * protos_ref[...] + a * mean
    out_ref[...] = jnp.where(cnt > 0.0, upd, protos_ref[...])


def kernel(x, support_examples, support_labels, num_shots, class_prototypes, prototype_counts):
    S = support_examples.shape[0]
    C, D = class_prototypes.shape
    flat = support_examples.reshape(S, D)
    labels3 = support_labels.reshape(S // _S_BLK, 1, _S_BLK)

    n_s = S // _S_BLK
    n_f = D // _F_BLK
    sums, counts = pl.pallas_call(
        _segsum_body,
        grid=(n_f, n_s),
        in_specs=[
            pl.BlockSpec((1, 1, _S_BLK), lambda f, s: (s, 0, 0)),
            pl.BlockSpec((_S_BLK, _F_BLK), lambda f, s: (s, f)),
        ],
        out_specs=[
            pl.BlockSpec((_C_PAD, _F_BLK), lambda f, s: (0, f)),
            pl.BlockSpec((_C_PAD, 128), lambda f, s: (0, 0)),
        ],
        out_shape=[
            jax.ShapeDtypeStruct((_C_PAD, D), jnp.float32),
            jax.ShapeDtypeStruct((_C_PAD, 128), jnp.float32),
        ],
    )(labels3, flat)

    blk = 200
    n_c = C // blk
    pc_b = jnp.broadcast_to(prototype_counts[:, None], (C, 128))
    new_protos = pl.pallas_call(
        _ema_body,
        grid=(n_c,),
        in_specs=[
            pl.BlockSpec((blk, D), lambda i: (i, 0)),
            pl.BlockSpec((blk, 128), lambda i: (i, 0)),
            pl.BlockSpec((blk, D), lambda i: (i, 0)),
            pl.BlockSpec((blk, 128), lambda i: (i, 0)),
        ],
        out_specs=pl.BlockSpec((blk, D), lambda i: (i, 0)),
        out_shape=jax.ShapeDtypeStruct((C, D), jnp.float32),
    )(sums, counts, class_prototypes, pc_b)

    return x, new_protos
